# Initial kernel scaffold; baseline (speedup 1.0000x reference)
import jax
import jax.numpy as jnp
from jax.experimental import pallas as pl


def kernel(node_input, edge_src, edge_dst, edge_attr, dist_embedding, W1, W2):
    raise NotImplementedError("write your pallas kernel here")


# speedup vs baseline: 1.1836x; 1.1836x over previous
import functools

import jax
import jax.numpy as jnp
from jax import lax
from jax.experimental import pallas as pl
from jax.experimental.pallas import tpu as pltpu
from jax.experimental.pallas import tpu_sc as plsc

N = 10000
E = 160000
D_NODE = 16
D_EDGE = 4
D_OUT = 8
D_EMB = 16
H = 64
SILU_NORM = 1.6790

NC = 2
NS = 16

G_WORKERS = NC * NS
G_REAL = E // G_WORKERS
G_CHUNK = 128
G_NCHUNK = 40
G_PAD = G_NCHUNK * G_CHUNK

_gather_mesh = plsc.VectorSubcoreMesh(core_axis_name="c", subcore_axis_name="s")
_SC_PARAMS = pltpu.CompilerParams(use_tc_tiling_on_sc=False)


@functools.partial(
    pl.kernel,
    out_type=jax.ShapeDtypeStruct((E, D_NODE), jnp.float32),
    mesh=_gather_mesh,
    compiler_params=_SC_PARAMS,
    scratch_types=[
        pltpu.VMEM((G_NCHUNK, G_CHUNK), jnp.int32),
        pltpu.VMEM((G_PAD, D_NODE), jnp.float32),
        pltpu.SemaphoreType.DMA,
    ],
)
def _gather_kernel(node_hbm, idx_hbm, out_hbm, idx_v, rows_v, sem):
    wid = lax.axis_index("s") * NC + lax.axis_index("c")
    pltpu.sync_copy(idx_hbm.at[wid], idx_v)

    def fire(j, carry):
        pltpu.make_async_copy(
            node_hbm.at[idx_v.at[j]],
            rows_v.at[pl.ds(j * G_CHUNK, G_CHUNK)],
            sem,
        ).start()
        return carry

    lax.fori_loop(0, G_NCHUNK, fire, 0)

    def drain(j, carry):
        pltpu.make_async_copy(
            node_hbm.at[idx_v.at[j]],
            rows_v.at[pl.ds(j * G_CHUNK, G_CHUNK)],
            sem,
        ).wait()
        return carry

    lax.fori_loop(0, G_NCHUNK, drain, 0)
    pltpu.sync_copy(rows_v.at[pl.ds(0, G_REAL)],
                    out_hbm.at[pl.ds(wid * G_REAL, G_REAL)])


B_EDGE = 2000


def _tc_body(demb_ref, attr_ref, xg_ref, w1_ref, w2_ref, out_ref):
    h = jnp.dot(demb_ref[...], w1_ref[...], preferred_element_type=jnp.float32)
    h = h * jax.nn.sigmoid(h) * SILU_NORM
    tpw = jnp.dot(h, w2_ref[...], preferred_element_type=jnp.float32)
    x = xg_ref[...]
    a = attr_ref[...]
    r = x[:, 0:1] * tpw[:, 0:D_EDGE * D_OUT]
    for i in range(1, D_NODE):
        r = r + x[:, i:i + 1] * tpw[:, 32 * i:32 * i + D_EDGE * D_OUT]
    ef = a[:, 0:1] * r[:, 0:D_OUT]
    for j in range(1, D_EDGE):
        ef = ef + a[:, j:j + 1] * r[:, 8 * j:8 * j + D_OUT]
    out_ref[...] = ef


def _tc_compute(demb, attr, xg, w1s, w2s):
    grid = (E // B_EDGE,)
    return pl.pallas_call(
        _tc_body,
        grid=grid,
        in_specs=[
            pl.BlockSpec((B_EDGE, D_EMB), lambda e: (e, 0)),
            pl.BlockSpec((B_EDGE, D_EDGE), lambda e: (e, 0)),
            pl.BlockSpec((B_EDGE, D_NODE), lambda e: (e, 0)),
            pl.BlockSpec((D_EMB, H), lambda e: (0, 0)),
            pl.BlockSpec((H, D_NODE * D_EDGE * D_OUT), lambda e: (0, 0)),
        ],
        out_specs=pl.BlockSpec((B_EDGE, D_OUT), lambda e: (e, 0)),
        out_shape=jax.ShapeDtypeStruct((E, D_OUT), jnp.float32),
        compiler_params=pltpu.CompilerParams(
            dimension_semantics=("arbitrary",),
        ),
    )(demb, attr, xg, w1s, w2s)


S_WORKERS = NS
S_REAL = E // S_WORKERS
S_CHUNK = 128
S_NCHUNK = 80
S_PAD = S_NCHUNK * S_CHUNK

_scatter_mesh = plsc.VectorSubcoreMesh(
    core_axis_name="c", subcore_axis_name="s", num_cores=1)


@functools.partial(
    pl.kernel,
    out_type=jax.ShapeDtypeStruct((N, D_OUT), jnp.float32),
    mesh=_scatter_mesh,
    compiler_params=_SC_PARAMS,
    scratch_types=[
        pltpu.VMEM((S_NCHUNK, S_CHUNK), jnp.int32),
        pltpu.VMEM((S_PAD, D_OUT), jnp.float32),
        pltpu.VMEM_SHARED((N, D_OUT), jnp.float32),
    ],
)
def _scatter_kernel(ef_hbm, idx_hbm, zeros_hbm, out_hbm, idx_v, ef_v, table):
    wid = lax.axis_index("s")
    pltpu.sync_copy(idx_hbm.at[wid], idx_v)
    pltpu.sync_copy(ef_hbm.at[pl.ds(wid * S_REAL, S_REAL)],
                    ef_v.at[pl.ds(0, S_REAL)])
    pltpu.sync_copy(zeros_hbm.at[pl.ds(0, S_PAD - S_REAL)],
                    ef_v.at[pl.ds(S_REAL, S_PAD - S_REAL)])

    @pl.when(wid == 0)
    def _init():
        pltpu.sync_copy(zeros_hbm, table)

    plsc.subcore_barrier()

    def body(j, carry):
        pltpu.sync_copy(ef_v.at[pl.ds(j * S_CHUNK, S_CHUNK)],
                        table.at[idx_v.at[j]], add=True)
        return carry

    lax.fori_loop(0, S_NCHUNK, body, 0)
    plsc.subcore_barrier()

    @pl.when(wid == 0)
    def _writeout():
        pltpu.sync_copy(table, out_hbm)


def _pad_indices(idx, workers, real, nchunk, chunk):
    idx = idx.astype(jnp.int32).reshape(workers, real)
    idx = jnp.pad(idx, ((0, 0), (0, nchunk * chunk - real)))
    return idx.reshape(workers, nchunk, chunk)


@jax.jit
def kernel(node_input, edge_src, edge_dst, edge_attr, dist_embedding, W1, W2):
    idx_dst = _pad_indices(edge_dst, G_WORKERS, G_REAL, G_NCHUNK, G_CHUNK)
    xg = _gather_kernel(node_input, idx_dst)

    w1s = W1 * (1.0 / jnp.sqrt(jnp.float32(D_EMB)))
    w2s = W2 * (1.0 / 256.0)
    ef = _tc_compute(dist_embedding, edge_attr, xg, w1s, w2s)

    idx_src = _pad_indices(edge_src, S_WORKERS, S_REAL, S_NCHUNK, S_CHUNK)
    zeros = jnp.zeros((N, D_OUT), jnp.float32)
    return _scatter_kernel(ef, idx_src, zeros)


# speedup vs baseline: 2.5955x; 2.1928x over previous
import functools

import jax
import jax.numpy as jnp
from jax import lax
from jax.experimental import pallas as pl
from jax.experimental.pallas import tpu as pltpu
from jax.experimental.pallas import tpu_sc as plsc

N = 10000
E = 160000
D_NODE = 16
D_EDGE = 4
D_OUT = 8
D_EMB = 16
H = 64
SILU_NORM = 1.6790

NC = 2
NS = 16

G_WORKERS = NC * NS
G_REAL = E // G_WORKERS
G_CHUNK = 128
G_NCHUNK = 40
G_PAD = G_NCHUNK * G_CHUNK

_gather_mesh = plsc.VectorSubcoreMesh(core_axis_name="c", subcore_axis_name="s")
_SC_PARAMS = pltpu.CompilerParams(use_tc_tiling_on_sc=False)


@functools.partial(
    pl.kernel,
    out_type=jax.ShapeDtypeStruct((E, D_NODE), jnp.float32),
    mesh=_gather_mesh,
    compiler_params=_SC_PARAMS,
    scratch_types=[
        pltpu.VMEM((G_NCHUNK, G_CHUNK), jnp.int32),
        pltpu.VMEM((G_PAD, D_NODE), jnp.float32),
        pltpu.SemaphoreType.DMA,
    ],
)
def _gather_kernel(node_hbm, idx_hbm, out_hbm, idx_v, rows_v, sem):
    wid = lax.axis_index("s") * NC + lax.axis_index("c")
    pltpu.sync_copy(idx_hbm.at[wid], idx_v)

    def fire(j, carry):
        pltpu.make_async_copy(
            node_hbm.at[idx_v.at[j]],
            rows_v.at[pl.ds(j * G_CHUNK, G_CHUNK)],
            sem,
        ).start()
        return carry

    lax.fori_loop(0, G_NCHUNK, fire, 0)

    def drain(j, carry):
        pltpu.make_async_copy(
            node_hbm.at[idx_v.at[j]],
            rows_v.at[pl.ds(j * G_CHUNK, G_CHUNK)],
            sem,
        ).wait()
        return carry

    lax.fori_loop(0, G_NCHUNK, drain, 0)
    pltpu.sync_copy(rows_v.at[pl.ds(0, G_REAL)],
                    out_hbm.at[pl.ds(wid * G_REAL, G_REAL)])


B_EDGE = 2000


CTOT = D_NODE * D_EDGE * D_OUT

_RX = jnp.asarray(jnp.repeat(jnp.eye(D_NODE, dtype=jnp.float32), 32, axis=1))
_RA = jnp.asarray(jnp.tile(
    jnp.repeat(jnp.eye(D_EDGE, dtype=jnp.float32), D_OUT, axis=1), (1, D_NODE)))
_SEL = jnp.asarray(jnp.tile(jnp.eye(D_OUT, dtype=jnp.float32),
                            (D_NODE * D_EDGE, 1)))


def _tc_body(demb_ref, attr_ref, xg_ref, w1_ref, w2_ref, rx_ref, ra_ref,
             sel_ref, out_ref):
    h = jnp.dot(demb_ref[...], w1_ref[...], preferred_element_type=jnp.float32)
    h = h * jax.nn.sigmoid(h) * SILU_NORM
    tpw = jnp.dot(h, w2_ref[...], preferred_element_type=jnp.float32)
    xr = jnp.dot(xg_ref[...], rx_ref[...], preferred_element_type=jnp.float32)
    ar = jnp.dot(attr_ref[...], ra_ref[...], preferred_element_type=jnp.float32)
    prod = tpw * (xr * ar)
    out_ref[...] = jnp.dot(prod, sel_ref[...],
                           preferred_element_type=jnp.float32)


def _tc_compute(demb, attr, xg, w1s, w2s):
    grid = (E // B_EDGE,)
    return pl.pallas_call(
        _tc_body,
        grid=grid,
        in_specs=[
            pl.BlockSpec((B_EDGE, D_EMB), lambda e: (e, 0)),
            pl.BlockSpec((B_EDGE, D_EDGE), lambda e: (e, 0)),
            pl.BlockSpec((B_EDGE, D_NODE), lambda e: (e, 0)),
            pl.BlockSpec((D_EMB, H), lambda e: (0, 0)),
            pl.BlockSpec((H, CTOT), lambda e: (0, 0)),
            pl.BlockSpec((D_NODE, CTOT), lambda e: (0, 0)),
            pl.BlockSpec((D_EDGE, CTOT), lambda e: (0, 0)),
            pl.BlockSpec((CTOT, D_OUT), lambda e: (0, 0)),
        ],
        out_specs=pl.BlockSpec((B_EDGE, D_OUT), lambda e: (e, 0)),
        out_shape=jax.ShapeDtypeStruct((E, D_OUT), jnp.float32),
        compiler_params=pltpu.CompilerParams(
            dimension_semantics=("arbitrary",),
        ),
    )(demb, attr, xg, w1s, w2s, _RX, _RA, _SEL)


S_WORKERS = NS
S_REAL = E // S_WORKERS
S_CHUNK = 128
S_NCHUNK = 80
S_PAD = S_NCHUNK * S_CHUNK

_scatter_mesh = plsc.VectorSubcoreMesh(
    core_axis_name="c", subcore_axis_name="s", num_cores=1)


@functools.partial(
    pl.kernel,
    out_type=jax.ShapeDtypeStruct((N, D_OUT), jnp.float32),
    mesh=_scatter_mesh,
    compiler_params=_SC_PARAMS,
    scratch_types=[
        pltpu.VMEM((S_NCHUNK, S_CHUNK), jnp.int32),
        pltpu.VMEM((S_PAD, D_OUT), jnp.float32),
        pltpu.VMEM_SHARED((N, D_OUT), jnp.float32),
    ],
)
def _scatter_kernel(ef_hbm, idx_hbm, zeros_hbm, out_hbm, idx_v, ef_v, table):
    wid = lax.axis_index("s")
    pltpu.sync_copy(idx_hbm.at[wid], idx_v)
    pltpu.sync_copy(ef_hbm.at[pl.ds(wid * S_REAL, S_REAL)],
                    ef_v.at[pl.ds(0, S_REAL)])
    pltpu.sync_copy(zeros_hbm.at[pl.ds(0, S_PAD - S_REAL)],
                    ef_v.at[pl.ds(S_REAL, S_PAD - S_REAL)])

    @pl.when(wid == 0)
    def _init():
        pltpu.sync_copy(zeros_hbm, table)

    plsc.subcore_barrier()

    def body(j, carry):
        pltpu.sync_copy(ef_v.at[pl.ds(j * S_CHUNK, S_CHUNK)],
                        table.at[idx_v.at[j]], add=True)
        return carry

    lax.fori_loop(0, S_NCHUNK, body, 0)
    plsc.subcore_barrier()

    @pl.when(wid == 0)
    def _writeout():
        pltpu.sync_copy(table, out_hbm)


def _pad_indices(idx, workers, real, nchunk, chunk):
    idx = idx.astype(jnp.int32).reshape(workers, real)
    idx = jnp.pad(idx, ((0, 0), (0, nchunk * chunk - real)))
    return idx.reshape(workers, nchunk, chunk)


@jax.jit
def kernel(node_input, edge_src, edge_dst, edge_attr, dist_embedding, W1, W2):
    idx_dst = _pad_indices(edge_dst, G_WORKERS, G_REAL, G_NCHUNK, G_CHUNK)
    xg = _gather_kernel(node_input, idx_dst)

    w1s = W1 * (1.0 / jnp.sqrt(jnp.float32(D_EMB)))
    w2s = W2 * (1.0 / 256.0)
    ef = _tc_compute(dist_embedding, edge_attr, xg, w1s, w2s)

    idx_src = _pad_indices(edge_src, S_WORKERS, S_REAL, S_NCHUNK, S_CHUNK)
    zeros = jnp.zeros((N, D_OUT), jnp.float32)
    return _scatter_kernel(ef, idx_src, zeros)


# speedup vs baseline: 2.6559x; 1.0233x over previous
import functools

import jax
import jax.numpy as jnp
import numpy as np
from jax import lax
from jax.experimental import pallas as pl
from jax.experimental.pallas import tpu as pltpu
from jax.experimental.pallas import tpu_sc as plsc

N = 10000
E = 160000
D_NODE = 16
D_EDGE = 4
D_OUT = 8
D_EMB = 16
H = 64
SILU_NORM = 1.6790

NC = 2
NS = 16

G_WORKERS = NC * NS
G_REAL = E // G_WORKERS
G_CHUNK = 128
G_NCHUNK = 40
G_PAD = G_NCHUNK * G_CHUNK

_gather_mesh = plsc.VectorSubcoreMesh(core_axis_name="c", subcore_axis_name="s")
_SC_PARAMS = pltpu.CompilerParams(use_tc_tiling_on_sc=False)


@functools.partial(
    pl.kernel,
    out_type=jax.ShapeDtypeStruct((E, D_NODE), jnp.float32),
    mesh=_gather_mesh,
    compiler_params=_SC_PARAMS,
    scratch_types=[
        pltpu.VMEM((G_NCHUNK, G_CHUNK), jnp.int32),
        pltpu.VMEM((G_PAD, D_NODE), jnp.float32),
        pltpu.SemaphoreType.DMA,
    ],
)
def _gather_kernel(node_hbm, idx_hbm, out_hbm, idx_v, rows_v, sem):
    wid = lax.axis_index("s") * NC + lax.axis_index("c")
    pltpu.sync_copy(idx_hbm.at[wid], idx_v)

    def fire(j, carry):
        pltpu.make_async_copy(
            node_hbm.at[idx_v.at[j]],
            rows_v.at[pl.ds(j * G_CHUNK, G_CHUNK)],
            sem,
        ).start()
        return carry

    lax.fori_loop(0, G_NCHUNK, fire, 0)

    def drain(j, carry):
        pltpu.make_async_copy(
            node_hbm.at[idx_v.at[j]],
            rows_v.at[pl.ds(j * G_CHUNK, G_CHUNK)],
            sem,
        ).wait()
        return carry

    lax.fori_loop(0, G_NCHUNK, drain, 0)
    pltpu.sync_copy(rows_v.at[pl.ds(0, G_REAL)],
                    out_hbm.at[pl.ds(wid * G_REAL, G_REAL)])


B_EDGE = 2000


CTOT = D_NODE * D_EDGE * D_OUT

_RX = np.repeat(np.eye(D_NODE, dtype=np.float32), 32, axis=1).astype(np.float32)
_RA = np.tile(
    np.repeat(np.eye(D_EDGE, dtype=np.float32), D_OUT, axis=1), (1, D_NODE))
_SEL = np.tile(np.eye(D_OUT, dtype=np.float32), (D_NODE * D_EDGE, 1))


def _tc_body(demb_ref, attr_ref, xg_ref, w1_ref, w2_ref, rx_ref, ra_ref,
             sel_ref, out_ref):
    h = jnp.dot(demb_ref[...], w1_ref[...], preferred_element_type=jnp.float32)
    h = h * jax.nn.sigmoid(h) * SILU_NORM
    tpw = jnp.dot(h.astype(jnp.bfloat16), w2_ref[...],
                  preferred_element_type=jnp.float32)
    xr = jnp.dot(xg_ref[...].astype(jnp.bfloat16), rx_ref[...],
                 preferred_element_type=jnp.float32)
    ar = jnp.dot(attr_ref[...], ra_ref[...], preferred_element_type=jnp.float32)
    prod = (tpw * (xr * ar)).astype(jnp.bfloat16)
    out_ref[...] = jnp.dot(prod, sel_ref[...],
                           preferred_element_type=jnp.float32)


def _tc_compute(demb, attr, xg, w1s, w2s):
    grid = (E // B_EDGE,)
    return pl.pallas_call(
        _tc_body,
        grid=grid,
        in_specs=[
            pl.BlockSpec((B_EDGE, D_EMB), lambda e: (e, 0)),
            pl.BlockSpec((B_EDGE, D_EDGE), lambda e: (e, 0)),
            pl.BlockSpec((B_EDGE, D_NODE), lambda e: (e, 0)),
            pl.BlockSpec((D_EMB, H), lambda e: (0, 0)),
            pl.BlockSpec((H, CTOT), lambda e: (0, 0)),
            pl.BlockSpec((D_NODE, CTOT), lambda e: (0, 0)),
            pl.BlockSpec((D_EDGE, CTOT), lambda e: (0, 0)),
            pl.BlockSpec((CTOT, D_OUT), lambda e: (0, 0)),
        ],
        out_specs=pl.BlockSpec((B_EDGE, D_OUT), lambda e: (e, 0)),
        out_shape=jax.ShapeDtypeStruct((E, D_OUT), jnp.float32),
        compiler_params=pltpu.CompilerParams(
            dimension_semantics=("arbitrary",),
        ),
    )(demb, attr, xg, w1s, w2s,
      _RX.astype(jnp.bfloat16), _RA.astype(jnp.bfloat16),
      _SEL.astype(jnp.bfloat16))


S_WORKERS = NS
S_REAL = E // S_WORKERS
S_CHUNK = 128
S_NCHUNK = 80
S_PAD = S_NCHUNK * S_CHUNK

_scatter_mesh = plsc.VectorSubcoreMesh(
    core_axis_name="c", subcore_axis_name="s", num_cores=1)


@functools.partial(
    pl.kernel,
    out_type=jax.ShapeDtypeStruct((N, D_OUT), jnp.float32),
    mesh=_scatter_mesh,
    compiler_params=_SC_PARAMS,
    scratch_types=[
        pltpu.VMEM((S_NCHUNK, S_CHUNK), jnp.int32),
        pltpu.VMEM((S_PAD, D_OUT), jnp.float32),
        pltpu.VMEM_SHARED((N, D_OUT), jnp.float32),
    ],
)
def _scatter_kernel(ef_hbm, idx_hbm, zeros_hbm, out_hbm, idx_v, ef_v, table):
    wid = lax.axis_index("s")
    pltpu.sync_copy(idx_hbm.at[wid], idx_v)
    pltpu.sync_copy(ef_hbm.at[pl.ds(wid * S_REAL, S_REAL)],
                    ef_v.at[pl.ds(0, S_REAL)])
    pltpu.sync_copy(zeros_hbm.at[pl.ds(0, S_PAD - S_REAL)],
                    ef_v.at[pl.ds(S_REAL, S_PAD - S_REAL)])

    @pl.when(wid == 0)
    def _init():
        pltpu.sync_copy(zeros_hbm, table)

    plsc.subcore_barrier()

    def body(j, carry):
        pltpu.sync_copy(ef_v.at[pl.ds(j * S_CHUNK, S_CHUNK)],
                        table.at[idx_v.at[j]], add=True)
        return carry

    lax.fori_loop(0, S_NCHUNK, body, 0)
    plsc.subcore_barrier()

    @pl.when(wid == 0)
    def _writeout():
        pltpu.sync_copy(table, out_hbm)


def _pad_indices(idx, workers, real, nchunk, chunk):
    idx = idx.astype(jnp.int32).reshape(workers, real)
    idx = jnp.pad(idx, ((0, 0), (0, nchunk * chunk - real)))
    return idx.reshape(workers, nchunk, chunk)


@jax.jit
def kernel(node_input, edge_src, edge_dst, edge_attr, dist_embedding, W1, W2):
    idx_dst = _pad_indices(edge_dst, G_WORKERS, G_REAL, G_NCHUNK, G_CHUNK)
    xg = _gather_kernel(node_input, idx_dst)

    w1s = (W1 * (1.0 / jnp.sqrt(jnp.float32(D_EMB)))).astype(jnp.bfloat16)
    w2s = (W2 * (1.0 / 256.0)).astype(jnp.bfloat16)
    ef = _tc_compute(dist_embedding.astype(jnp.bfloat16),
                     edge_attr.astype(jnp.bfloat16), xg, w1s, w2s)

    idx_src = _pad_indices(edge_src, S_WORKERS, S_REAL, S_NCHUNK, S_CHUNK)
    zeros = jnp.zeros((N, D_OUT), jnp.float32)
    return _scatter_kernel(ef, idx_src, zeros)
